# baseline (speedup 1.0000x reference)
import functools

import jax
import jax.numpy as jnp
import numpy as np
from jax import lax
from jax.experimental import pallas as pl
from jax.experimental.pallas import tpu as pltpu
from jax.experimental.pallas import tpu_sc as plsc

K = 16
QCHUNK = 128
PSLICE = 128
QBLK = 64
NW = 32


def _ceil_to(v, m):
    return ((v + m - 1) // m) * m


def _chunk_of(per_w):
    for c in range(128, 0, -8):
        if per_w % c == 0:
            return c
    return per_w


def _sc_row_gather(n_rows, d, n_idx, dtype):
    per_w = n_idx // NW
    chk = _chunk_of(per_w)
    n_loops = per_w // chk
    mesh = plsc.VectorSubcoreMesh(core_axis_name="c", subcore_axis_name="s")

    @functools.partial(
        pl.kernel,
        mesh=mesh,
        compiler_params=pltpu.CompilerParams(use_tc_tiling_on_sc=False),
        out_type=jax.ShapeDtypeStruct((n_idx, d), dtype),
        scratch_types=[
            pltpu.VMEM((chk,), jnp.int32),
            pltpu.VMEM((chk, d), dtype),
            pltpu.SemaphoreType.DMA,
        ],
    )
    def gather_k(table_hbm, idx_hbm, out_hbm, idx_v, rows_v, sem):
        wid = lax.axis_index("s") * 2 + lax.axis_index("c")
        base = wid * per_w

        def body(j, carry):
            off = base + j * chk
            pltpu.sync_copy(idx_hbm.at[pl.ds(off, chk)], idx_v)
            pltpu.async_copy(table_hbm.at[idx_v], rows_v, sem).wait()
            pltpu.sync_copy(rows_v, out_hbm.at[pl.ds(off, chk)])
            return carry

        lax.fori_loop(0, n_loops, body, 0)

    return gather_k


def _sc_edge_gather(n_rows, d1, d2, n_idx):
    per_w = n_idx // NW
    chk = _chunk_of(per_w)
    n_loops = per_w // chk
    mesh = plsc.VectorSubcoreMesh(core_axis_name="c", subcore_axis_name="s")

    @functools.partial(
        pl.kernel,
        mesh=mesh,
        compiler_params=pltpu.CompilerParams(use_tc_tiling_on_sc=False),
        out_type=(
            jax.ShapeDtypeStruct((n_idx, d1), jnp.float32),
            jax.ShapeDtypeStruct((n_idx, d2), jnp.float32),
        ),
        scratch_types=[
            pltpu.VMEM((chk,), jnp.int32),
            pltpu.VMEM((chk, d1), jnp.float32),
            pltpu.VMEM((chk, d2), jnp.float32),
            pltpu.SemaphoreType.DMA,
            pltpu.SemaphoreType.DMA,
        ],
    )
    def gather_k(t1_hbm, t2_hbm, idx_hbm, o1_hbm, o2_hbm,
                 idx_v, r1_v, r2_v, sem1, sem2):
        wid = lax.axis_index("s") * 2 + lax.axis_index("c")
        base = wid * per_w

        def body(j, carry):
            off = base + j * chk
            pltpu.sync_copy(idx_hbm.at[pl.ds(off, chk)], idx_v)
            cp1 = pltpu.async_copy(t1_hbm.at[idx_v], r1_v, sem1)
            cp2 = pltpu.async_copy(t2_hbm.at[idx_v], r2_v, sem2)
            cp1.wait()
            cp2.wait()
            pltpu.sync_copy(r1_v, o1_hbm.at[pl.ds(off, chk)])
            pltpu.sync_copy(r2_v, o2_hbm.at[pl.ds(off, chk)])
            return carry

        lax.fori_loop(0, n_loops, body, 0)

    return gather_k


def _knn_body(n_groups, posq_ref, posb_ref, pn_ref, nbr_ref):
    qp = posq_ref[...]
    qn = jnp.sum(qp * qp, axis=1, keepdims=True)
    qpb = qp.astype(jnp.bfloat16)
    imax = jnp.int32(0x7FFFFFFF)
    full = jnp.full((QCHUNK, PSLICE), imax, jnp.int32)

    def g_body(g, carry):
        a1, a2, a3 = carry
        dsb = posb_ref[pl.ds(g * PSLICE, PSLICE), :]
        dot = lax.dot_general(
            qpb, dsb, (((1,), (1,)), ((), ())),
            preferred_element_type=jnp.float32,
        )
        d2 = (qn + pn_ref[pl.ds(g, 1), :]) - 2.0 * dot
        bits = lax.bitcast_convert_type(d2, jnp.int32)
        key = (bits & jnp.int32(~511)) | g.astype(jnp.int32)
        t = jnp.maximum(a1, key)
        a1 = jnp.minimum(a1, key)
        t2 = jnp.maximum(a2, t)
        a2 = jnp.minimum(a2, t)
        a3 = jnp.minimum(a3, t2)
        return a1, a2, a3

    a1, a2, a3 = lax.fori_loop(0, n_groups, g_body, (full, full, full))

    lane_b = lax.broadcasted_iota(jnp.int32, (QCHUNK, PSLICE), 1)
    lane16 = lax.broadcasted_iota(jnp.int32, (QCHUNK, K), 1)

    def e_body(k, carry):
        ws, n1, n2, nbr = carry
        m = jnp.min(ws, axis=1, keepdims=True)
        j = jnp.min(jnp.where(ws == m, lane_b, PSLICE), axis=1, keepdims=True)
        sel = lane_b == j
        ws = jnp.where(sel, n1, ws)
        n1 = jnp.where(sel, n2, n1)
        n2 = jnp.where(sel, imax, n2)
        point = (m & 511) * PSLICE + j
        nbr = jnp.where(lane16 == k, point, nbr)
        return ws, n1, n2, nbr

    _, _, _, nbr = lax.fori_loop(
        0, K, e_body, (a1, a2, a3, jnp.zeros((QCHUNK, K), jnp.int32))
    )
    nbr_ref[...] = nbr


def _knn_call(posq16, posb, pnr, mp, np_pad):
    n_groups = np_pad // PSLICE
    grid = mp // QCHUNK
    return pl.pallas_call(
        functools.partial(_knn_body, n_groups),
        grid=(grid,),
        in_specs=[
            pl.BlockSpec((QCHUNK, 16), lambda i: (i, 0)),
            pl.BlockSpec((np_pad, 16), lambda i: (0, 0)),
            pl.BlockSpec((n_groups, PSLICE), lambda i: (0, 0)),
        ],
        out_specs=pl.BlockSpec((QCHUNK, K), lambda i: (i, 0)),
        out_shape=jax.ShapeDtypeStruct((mp, K), jnp.int32),
    )(posq16, posb, pnr)


def _msg_body(xj_ref, pj_ref, pq_ref, rmat_ref, pmat_ref, a_ref, b_ref,
              wpb_ref, watt_ref, batt_ref, wglob_ref, bglob_ref, out_ref):
    eb = QBLK * K
    pj = pj_ref[...]
    pq = pq_ref[...]
    rmat = rmat_ref[...]
    pos_i = jnp.dot(rmat, pq, preferred_element_type=jnp.float32)
    vij = pos_i - pj
    d2 = jnp.sum(vij * vij, axis=1, keepdims=True)
    dij = jnp.sqrt(d2 + 1e-12)
    w4 = wpb_ref[0:1, :]
    bpos = wpb_ref[1:2, :]
    rij = (
        jnp.dot(pos_i, a_ref[...], preferred_element_type=jnp.float32)
        + jnp.dot(pj, b_ref[...], preferred_element_type=jnp.float32)
        + dij * w4
        + bpos
    )
    fij = jnp.concatenate([xj_ref[...], rij], axis=1)
    g = jnp.dot(fij, watt_ref[...], preferred_element_type=jnp.float32)
    g = g + batt_ref[0:1, :]
    m = jnp.max(g, axis=1, keepdims=True)
    e = jnp.exp(g - m)
    s = e / jnp.sum(e, axis=1, keepdims=True)
    msg = s * fij
    aggr = jnp.dot(pmat_ref[...], msg, preferred_element_type=jnp.float32)
    out = jnp.dot(aggr, wglob_ref[...], preferred_element_type=jnp.float32)
    out_ref[...] = out + bglob_ref[0:1, :]


def _msg_call(xj, pj, pq, rmat, pmat, a16, b16, wpb, watt, battp, wglob,
              bglobp, mp, d_feat, d_f, d_out):
    eb = QBLK * K
    grid = mp // QBLK
    full = lambda shape: pl.BlockSpec(shape, lambda i: tuple(0 for _ in shape))
    return pl.pallas_call(
        _msg_body,
        grid=(grid,),
        in_specs=[
            pl.BlockSpec((eb, d_feat), lambda i: (i, 0)),
            pl.BlockSpec((eb, 16), lambda i: (i, 0)),
            pl.BlockSpec((QBLK, 16), lambda i: (i, 0)),
            full((eb, QBLK)),
            full((QBLK, eb)),
            full((16, 64)),
            full((16, 64)),
            full((8, 64)),
            full((d_f, d_f)),
            full((8, d_f)),
            full((d_f, d_out)),
            full((8, d_out)),
        ],
        out_specs=pl.BlockSpec((QBLK, d_out), lambda i: (i, 0)),
        out_shape=jax.ShapeDtypeStruct((mp, d_out), jnp.float32),
    )(xj, pj, pq, rmat, pmat, a16, b16, wpb, watt, battp, wglob, bglobp)


def kernel(x, pos, batch, W_pos, b_pos, W_att, b_att, W_glob, b_glob):
    n, d_feat = x.shape
    m = n // 4
    d_r = W_pos.shape[1]
    d_f = d_feat + d_r
    d_out = W_glob.shape[1]

    np_pad = _ceil_to(n, PSLICE)
    mp = _ceil_to(m, QCHUNK)
    e_pad = mp * K

    idx = jax.random.randint(jax.random.key(42), (m,), 0, n)
    idx_p = jnp.concatenate([idx, jnp.zeros((mp - m,), jnp.int32)])

    pos16 = jnp.concatenate([pos, jnp.zeros((n, 13), jnp.float32)], axis=1)
    pad_rows = jnp.concatenate(
        [
            jnp.full((np_pad - n, 1), 1e18, jnp.float32),
            jnp.zeros((np_pad - n, 15), jnp.float32),
        ],
        axis=1,
    )
    pos16 = jnp.concatenate([pos16, pad_rows], axis=0)

    posq16 = _sc_row_gather(np_pad, 16, mp, jnp.float32)(pos16, idx_p)

    pnr = jnp.sum(pos16 * pos16, axis=1).reshape(np_pad // PSLICE, PSLICE)
    posb = pos16.astype(jnp.bfloat16)
    nbr = _knn_call(posq16, posb, pnr, mp, np_pad)
    nbr_flat = jnp.minimum(nbr.reshape(e_pad), n - 1)

    xj, pj = _sc_edge_gather(np_pad, d_feat, 16, e_pad)(
        jnp.concatenate([x, jnp.zeros((np_pad - n, d_feat), x.dtype)]),
        pos16, nbr_flat)

    rep = np.zeros((QBLK * K, QBLK), np.float32)
    rep[np.arange(QBLK * K), np.arange(QBLK * K) // K] = 1.0
    rmat = jnp.asarray(rep)
    pmat = jnp.asarray(rep.T / K)

    a16 = jnp.zeros((16, d_r), jnp.float32).at[:3].set(W_pos[0:3] + W_pos[6:9])
    b16 = jnp.zeros((16, d_r), jnp.float32).at[:3].set(W_pos[3:6] - W_pos[6:9])
    wpb = jnp.zeros((8, d_r), jnp.float32).at[0].set(W_pos[9]).at[1].set(b_pos)
    battp = jnp.zeros((8, d_f), jnp.float32).at[0].set(b_att)
    bglobp = jnp.zeros((8, d_out), jnp.float32).at[0].set(b_glob)

    out = _msg_call(xj, pj, posq16, rmat, pmat, a16, b16, wpb, W_att, battp,
                    W_glob, bglobp, mp, d_feat, d_f, d_out)
    return out[:m]


# speedup vs baseline: 3.2079x; 3.2079x over previous
import functools

import jax
import jax.numpy as jnp
import numpy as np
from jax import lax
from jax.experimental import pallas as pl
from jax.experimental.pallas import tpu as pltpu
from jax.experimental.pallas import tpu_sc as plsc

K = 16
QCHUNK = 256
PSLICE = 1024
TOPB = 2
QBLK = 64
NW = 32


def _ceil_to(v, m):
    return ((v + m - 1) // m) * m


def _chunk_of(per_w):
    for c in range(128, 0, -8):
        if per_w % c == 0:
            return c
    return per_w


def _sc_row_gather(n_rows, d, n_idx, dtype):
    per_w = n_idx // NW
    chk = _chunk_of(per_w)
    n_loops = per_w // chk
    mesh = plsc.VectorSubcoreMesh(core_axis_name="c", subcore_axis_name="s")

    @functools.partial(
        pl.kernel,
        mesh=mesh,
        compiler_params=pltpu.CompilerParams(use_tc_tiling_on_sc=False),
        out_type=jax.ShapeDtypeStruct((n_idx, d), dtype),
        scratch_types=[
            pltpu.VMEM((chk,), jnp.int32),
            pltpu.VMEM((chk, d), dtype),
            pltpu.SemaphoreType.DMA,
        ],
    )
    def gather_k(table_hbm, idx_hbm, out_hbm, idx_v, rows_v, sem):
        wid = lax.axis_index("s") * 2 + lax.axis_index("c")
        base = wid * per_w

        def body(j, carry):
            off = base + j * chk
            pltpu.sync_copy(idx_hbm.at[pl.ds(off, chk)], idx_v)
            pltpu.async_copy(table_hbm.at[idx_v], rows_v, sem).wait()
            pltpu.sync_copy(rows_v, out_hbm.at[pl.ds(off, chk)])
            return carry

        lax.fori_loop(0, n_loops, body, 0)

    return gather_k


def _sc_edge_gather(n_rows, d1, d2, n_idx):
    per_w = n_idx // NW
    chk = _chunk_of(per_w)
    n_loops = per_w // chk
    mesh = plsc.VectorSubcoreMesh(core_axis_name="c", subcore_axis_name="s")

    @functools.partial(
        pl.kernel,
        mesh=mesh,
        compiler_params=pltpu.CompilerParams(use_tc_tiling_on_sc=False),
        out_type=(
            jax.ShapeDtypeStruct((n_idx, d1), jnp.float32),
            jax.ShapeDtypeStruct((n_idx, d2), jnp.float32),
        ),
        scratch_types=[
            pltpu.VMEM((chk,), jnp.int32),
            pltpu.VMEM((chk, d1), jnp.float32),
            pltpu.VMEM((chk, d2), jnp.float32),
            pltpu.SemaphoreType.DMA,
            pltpu.SemaphoreType.DMA,
        ],
    )
    def gather_k(t1_hbm, t2_hbm, idx_hbm, o1_hbm, o2_hbm,
                 idx_v, r1_v, r2_v, sem1, sem2):
        wid = lax.axis_index("s") * 2 + lax.axis_index("c")
        base = wid * per_w

        def body(j, carry):
            off = base + j * chk
            pltpu.sync_copy(idx_hbm.at[pl.ds(off, chk)], idx_v)
            cp1 = pltpu.async_copy(t1_hbm.at[idx_v], r1_v, sem1)
            cp2 = pltpu.async_copy(t2_hbm.at[idx_v], r2_v, sem2)
            cp1.wait()
            cp2.wait()
            pltpu.sync_copy(r1_v, o1_hbm.at[pl.ds(off, chk)])
            pltpu.sync_copy(r2_v, o2_hbm.at[pl.ds(off, chk)])
            return carry

        lax.fori_loop(0, n_loops, body, 0)

    return gather_k


def _knn_body(n_groups, n_cand, posq_ref, posb_ref, pn_ref, nbr_ref, top_ref):
    qp = posq_ref[...]
    qn = jnp.sum(qp * qp, axis=1, keepdims=True)
    qpb = qp.astype(jnp.bfloat16)
    imax = jnp.int32(0x7FFFFFFF)

    for r in range(TOPB):
        top_ref[r] = jnp.full((QCHUNK, PSLICE), imax, jnp.int32)

    def g_body(g, carry):
        dsb = posb_ref[pl.ds(g * PSLICE, PSLICE), :]
        dot = lax.dot_general(
            qpb, dsb, (((1,), (1,)), ((), ())),
            preferred_element_type=jnp.float32,
        )
        d2 = (qn + pn_ref[pl.ds(g, 1), :]) - 2.0 * dot
        bits = lax.bitcast_convert_type(d2, jnp.int32)
        key = (bits & jnp.int32(~63)) | g.astype(jnp.int32)
        a1, a2 = top_ref[0], top_ref[1]
        t = jnp.maximum(a1, key)
        top_ref[0] = jnp.minimum(a1, key)
        top_ref[1] = jnp.minimum(a2, t)
        return carry

    lax.fori_loop(0, n_groups, g_body, 0)

    lane_b = lax.broadcasted_iota(jnp.int32, (QCHUNK, PSLICE), 1)
    lane16 = lax.broadcasted_iota(jnp.int32, (QCHUNK, K), 1)

    def e_body(k, carry):
        ws, n1, nbr = carry
        m = jnp.min(ws, axis=1, keepdims=True)
        j = jnp.min(jnp.where(ws == m, lane_b, PSLICE), axis=1, keepdims=True)
        sel = lane_b == j
        ws = jnp.where(sel, n1, ws)
        n1 = jnp.where(sel, imax, n1)
        point = (m & 63) * PSLICE + j
        nbr = jnp.where(lane16 == k, point, nbr)
        return ws, n1, nbr

    _, _, nbr = lax.fori_loop(
        0, K, e_body,
        (top_ref[0], top_ref[1], jnp.zeros((QCHUNK, K), jnp.int32)),
    )
    nbr_ref[...] = nbr


def _knn_call(posq16, posb, pnr, mp, np_pad):
    n_groups = np_pad // PSLICE
    n_cand = TOPB * PSLICE
    grid = mp // QCHUNK
    return pl.pallas_call(
        functools.partial(_knn_body, n_groups, n_cand),
        grid=(grid,),
        in_specs=[
            pl.BlockSpec((QCHUNK, 16), lambda i: (i, 0)),
            pl.BlockSpec((np_pad, 16), lambda i: (0, 0)),
            pl.BlockSpec((n_groups, PSLICE), lambda i: (0, 0)),
        ],
        out_specs=pl.BlockSpec((QCHUNK, K), lambda i: (i, 0)),
        out_shape=jax.ShapeDtypeStruct((mp, K), jnp.int32),
        scratch_shapes=[pltpu.VMEM((TOPB, QCHUNK, PSLICE), jnp.int32)],
    )(posq16, posb, pnr)


def _msg_body(xj_ref, pj_ref, pq_ref, rmat_ref, pmat_ref, a_ref, b_ref,
              wpb_ref, watt_ref, batt_ref, wglob_ref, bglob_ref, out_ref):
    eb = QBLK * K
    pj = pj_ref[...]
    pq = pq_ref[...]
    rmat = rmat_ref[...]
    pos_i = jnp.dot(rmat, pq, preferred_element_type=jnp.float32)
    vij = pos_i - pj
    d2 = jnp.sum(vij * vij, axis=1, keepdims=True)
    dij = jnp.sqrt(d2 + 1e-12)
    w4 = wpb_ref[0:1, :]
    bpos = wpb_ref[1:2, :]
    rij = (
        jnp.dot(pos_i, a_ref[...], preferred_element_type=jnp.float32)
        + jnp.dot(pj, b_ref[...], preferred_element_type=jnp.float32)
        + dij * w4
        + bpos
    )
    fij = jnp.concatenate([xj_ref[...], rij], axis=1)
    g = jnp.dot(fij, watt_ref[...], preferred_element_type=jnp.float32)
    g = g + batt_ref[0:1, :]
    m = jnp.max(g, axis=1, keepdims=True)
    e = jnp.exp(g - m)
    s = e / jnp.sum(e, axis=1, keepdims=True)
    msg = s * fij
    aggr = jnp.dot(pmat_ref[...], msg, preferred_element_type=jnp.float32)
    out = jnp.dot(aggr, wglob_ref[...], preferred_element_type=jnp.float32)
    out_ref[...] = out + bglob_ref[0:1, :]


def _msg_call(xj, pj, pq, rmat, pmat, a16, b16, wpb, watt, battp, wglob,
              bglobp, mp, d_feat, d_f, d_out):
    eb = QBLK * K
    grid = mp // QBLK
    full = lambda shape: pl.BlockSpec(shape, lambda i: tuple(0 for _ in shape))
    return pl.pallas_call(
        _msg_body,
        grid=(grid,),
        in_specs=[
            pl.BlockSpec((eb, d_feat), lambda i: (i, 0)),
            pl.BlockSpec((eb, 16), lambda i: (i, 0)),
            pl.BlockSpec((QBLK, 16), lambda i: (i, 0)),
            full((eb, QBLK)),
            full((QBLK, eb)),
            full((16, 64)),
            full((16, 64)),
            full((8, 64)),
            full((d_f, d_f)),
            full((8, d_f)),
            full((d_f, d_out)),
            full((8, d_out)),
        ],
        out_specs=pl.BlockSpec((QBLK, d_out), lambda i: (i, 0)),
        out_shape=jax.ShapeDtypeStruct((mp, d_out), jnp.float32),
    )(xj, pj, pq, rmat, pmat, a16, b16, wpb, watt, battp, wglob, bglobp)


def kernel(x, pos, batch, W_pos, b_pos, W_att, b_att, W_glob, b_glob):
    n, d_feat = x.shape
    m = n // 4
    d_r = W_pos.shape[1]
    d_f = d_feat + d_r
    d_out = W_glob.shape[1]

    np_pad = _ceil_to(n, PSLICE)
    mp = _ceil_to(m, QCHUNK)
    e_pad = mp * K

    idx = jax.random.randint(jax.random.key(42), (m,), 0, n)
    idx_p = jnp.concatenate([idx, jnp.zeros((mp - m,), jnp.int32)])

    pos16 = jnp.concatenate([pos, jnp.zeros((n, 13), jnp.float32)], axis=1)
    pad_rows = jnp.concatenate(
        [
            jnp.full((np_pad - n, 1), 1e18, jnp.float32),
            jnp.zeros((np_pad - n, 15), jnp.float32),
        ],
        axis=1,
    )
    pos16 = jnp.concatenate([pos16, pad_rows], axis=0)

    posq16 = _sc_row_gather(np_pad, 16, mp, jnp.float32)(pos16, idx_p)

    pnr = jnp.sum(pos16 * pos16, axis=1).reshape(np_pad // PSLICE, PSLICE)
    posb = pos16.astype(jnp.bfloat16)
    nbr = _knn_call(posq16, posb, pnr, mp, np_pad)
    nbr_flat = jnp.minimum(nbr.reshape(e_pad), n - 1)

    xj, pj = _sc_edge_gather(np_pad, d_feat, 16, e_pad)(
        jnp.concatenate([x, jnp.zeros((np_pad - n, d_feat), x.dtype)]),
        pos16, nbr_flat)

    rep = np.zeros((QBLK * K, QBLK), np.float32)
    rep[np.arange(QBLK * K), np.arange(QBLK * K) // K] = 1.0
    rmat = jnp.asarray(rep)
    pmat = jnp.asarray(rep.T / K)

    a16 = jnp.zeros((16, d_r), jnp.float32).at[:3].set(W_pos[0:3] + W_pos[6:9])
    b16 = jnp.zeros((16, d_r), jnp.float32).at[:3].set(W_pos[3:6] - W_pos[6:9])
    wpb = jnp.zeros((8, d_r), jnp.float32).at[0].set(W_pos[9]).at[1].set(b_pos)
    battp = jnp.zeros((8, d_f), jnp.float32).at[0].set(b_att)
    bglobp = jnp.zeros((8, d_out), jnp.float32).at[0].set(b_glob)

    out = _msg_call(xj, pj, posq16, rmat, pmat, a16, b16, wpb, W_att, battp,
                    W_glob, bglobp, mp, d_feat, d_f, d_out)
    return out[:m]


# speedup vs baseline: 3.5313x; 1.1008x over previous
import functools

import jax
import jax.numpy as jnp
import numpy as np
from jax import lax
from jax.experimental import pallas as pl
from jax.experimental.pallas import tpu as pltpu
from jax.experimental.pallas import tpu_sc as plsc

K = 16
QCHUNK = 512
PSLICE = 1024
TOPB = 2
QBLK = 64
NW = 32


def _ceil_to(v, m):
    return ((v + m - 1) // m) * m


def _chunk_of(per_w):
    for c in range(128, 0, -8):
        if per_w % c == 0:
            return c
    return per_w


def _sc_row_gather(n_rows, d, n_idx, dtype):
    per_w = n_idx // NW
    chk = _chunk_of(per_w)
    n_loops = per_w // chk
    mesh = plsc.VectorSubcoreMesh(core_axis_name="c", subcore_axis_name="s")

    @functools.partial(
        pl.kernel,
        mesh=mesh,
        compiler_params=pltpu.CompilerParams(use_tc_tiling_on_sc=False),
        out_type=jax.ShapeDtypeStruct((n_idx, d), dtype),
        scratch_types=[
            pltpu.VMEM((chk,), jnp.int32),
            pltpu.VMEM((chk, d), dtype),
            pltpu.SemaphoreType.DMA,
        ],
    )
    def gather_k(table_hbm, idx_hbm, out_hbm, idx_v, rows_v, sem):
        wid = lax.axis_index("s") * 2 + lax.axis_index("c")
        base = wid * per_w

        def body(j, carry):
            off = base + j * chk
            pltpu.sync_copy(idx_hbm.at[pl.ds(off, chk)], idx_v)
            pltpu.async_copy(table_hbm.at[idx_v], rows_v, sem).wait()
            pltpu.sync_copy(rows_v, out_hbm.at[pl.ds(off, chk)])
            return carry

        lax.fori_loop(0, n_loops, body, 0)

    return gather_k


def _sc_edge_gather(n_rows, d1, d2, n_idx):
    per_w = n_idx // NW
    chk = _chunk_of(per_w)
    n_loops = per_w // chk
    mesh = plsc.VectorSubcoreMesh(core_axis_name="c", subcore_axis_name="s")

    @functools.partial(
        pl.kernel,
        mesh=mesh,
        compiler_params=pltpu.CompilerParams(use_tc_tiling_on_sc=False),
        out_type=(
            jax.ShapeDtypeStruct((n_idx, d1), jnp.float32),
            jax.ShapeDtypeStruct((n_idx, d2), jnp.float32),
        ),
        scratch_types=[
            pltpu.VMEM((chk,), jnp.int32),
            pltpu.VMEM((chk, d1), jnp.float32),
            pltpu.VMEM((chk, d2), jnp.float32),
            pltpu.SemaphoreType.DMA,
            pltpu.SemaphoreType.DMA,
        ],
    )
    def gather_k(t1_hbm, t2_hbm, idx_hbm, o1_hbm, o2_hbm,
                 idx_v, r1_v, r2_v, sem1, sem2):
        wid = lax.axis_index("s") * 2 + lax.axis_index("c")
        base = wid * per_w

        def body(j, carry):
            off = base + j * chk
            pltpu.sync_copy(idx_hbm.at[pl.ds(off, chk)], idx_v)
            cp1 = pltpu.async_copy(t1_hbm.at[idx_v], r1_v, sem1)
            cp2 = pltpu.async_copy(t2_hbm.at[idx_v], r2_v, sem2)
            cp1.wait()
            cp2.wait()
            pltpu.sync_copy(r1_v, o1_hbm.at[pl.ds(off, chk)])
            pltpu.sync_copy(r2_v, o2_hbm.at[pl.ds(off, chk)])
            return carry

        lax.fori_loop(0, n_loops, body, 0)

    return gather_k


def _knn_body(n_groups, n_cand, posq_ref, posb_ref, pn_ref, nbr_ref, top_ref):
    qp = posq_ref[...]
    qn = jnp.sum(qp * qp, axis=1, keepdims=True)
    qpb = qp.astype(jnp.bfloat16)
    imax = jnp.int32(0x7FFFFFFF)

    for r in range(TOPB):
        top_ref[r] = jnp.full((QCHUNK, PSLICE), imax, jnp.int32)

    def g_body(g, carry):
        dsb = posb_ref[pl.ds(g * PSLICE, PSLICE), :]
        dot = lax.dot_general(
            qpb, dsb, (((1,), (1,)), ((), ())),
            preferred_element_type=jnp.float32,
        )
        d2 = (qn + pn_ref[pl.ds(g, 1), :]) - 2.0 * dot
        bits = lax.bitcast_convert_type(d2, jnp.int32)
        key = (bits & jnp.int32(~63)) | g.astype(jnp.int32)
        a1, a2 = top_ref[0], top_ref[1]
        t = jnp.maximum(a1, key)
        top_ref[0] = jnp.minimum(a1, key)
        top_ref[1] = jnp.minimum(a2, t)
        return carry

    lax.fori_loop(0, n_groups, g_body, 0)

    lane_b = lax.broadcasted_iota(jnp.int32, (QCHUNK, PSLICE), 1)
    lane16 = lax.broadcasted_iota(jnp.int32, (QCHUNK, K), 1)

    def e_body(k, carry):
        ws, n1, nbr = carry
        m = jnp.min(ws, axis=1, keepdims=True)
        j = jnp.min(jnp.where(ws == m, lane_b, PSLICE), axis=1, keepdims=True)
        sel = lane_b == j
        ws = jnp.where(sel, n1, ws)
        n1 = jnp.where(sel, imax, n1)
        point = (m & 63) * PSLICE + j
        nbr = jnp.where(lane16 == k, point, nbr)
        return ws, n1, nbr

    _, _, nbr = lax.fori_loop(
        0, K, e_body,
        (top_ref[0], top_ref[1], jnp.zeros((QCHUNK, K), jnp.int32)),
    )
    nbr_ref[...] = nbr


def _knn_call(posq16, posb, pnr, mp, np_pad):
    n_groups = np_pad // PSLICE
    n_cand = TOPB * PSLICE
    grid = mp // QCHUNK
    return pl.pallas_call(
        functools.partial(_knn_body, n_groups, n_cand),
        grid=(grid,),
        in_specs=[
            pl.BlockSpec((QCHUNK, 16), lambda i: (i, 0)),
            pl.BlockSpec((np_pad, 16), lambda i: (0, 0)),
            pl.BlockSpec((n_groups, PSLICE), lambda i: (0, 0)),
        ],
        out_specs=pl.BlockSpec((QCHUNK, K), lambda i: (i, 0)),
        out_shape=jax.ShapeDtypeStruct((mp, K), jnp.int32),
        scratch_shapes=[pltpu.VMEM((TOPB, QCHUNK, PSLICE), jnp.int32)],
    )(posq16, posb, pnr)


def _msg_body(xj_ref, pj_ref, pq_ref, rmat_ref, pmat_ref, a_ref, b_ref,
              wpb_ref, watt_ref, batt_ref, wglob_ref, bglob_ref, out_ref):
    eb = QBLK * K
    pj = pj_ref[...]
    pq = pq_ref[...]
    rmat = rmat_ref[...]
    pos_i = jnp.dot(rmat, pq, preferred_element_type=jnp.float32)
    vij = pos_i - pj
    d2 = jnp.sum(vij * vij, axis=1, keepdims=True)
    dij = jnp.sqrt(d2 + 1e-12)
    w4 = wpb_ref[0:1, :]
    bpos = wpb_ref[1:2, :]
    rij = (
        jnp.dot(pos_i, a_ref[...], preferred_element_type=jnp.float32)
        + jnp.dot(pj, b_ref[...], preferred_element_type=jnp.float32)
        + dij * w4
        + bpos
    )
    fij = jnp.concatenate([xj_ref[...], rij], axis=1)
    g = jnp.dot(fij, watt_ref[...], preferred_element_type=jnp.float32)
    g = g + batt_ref[0:1, :]
    m = jnp.max(g, axis=1, keepdims=True)
    e = jnp.exp(g - m)
    s = e / jnp.sum(e, axis=1, keepdims=True)
    msg = s * fij
    aggr = jnp.dot(pmat_ref[...], msg, preferred_element_type=jnp.float32)
    out = jnp.dot(aggr, wglob_ref[...], preferred_element_type=jnp.float32)
    out_ref[...] = out + bglob_ref[0:1, :]


def _msg_call(xj, pj, pq, rmat, pmat, a16, b16, wpb, watt, battp, wglob,
              bglobp, mp, d_feat, d_f, d_out):
    eb = QBLK * K
    grid = mp // QBLK
    full = lambda shape: pl.BlockSpec(shape, lambda i: tuple(0 for _ in shape))
    return pl.pallas_call(
        _msg_body,
        grid=(grid,),
        in_specs=[
            pl.BlockSpec((eb, d_feat), lambda i: (i, 0)),
            pl.BlockSpec((eb, 16), lambda i: (i, 0)),
            pl.BlockSpec((QBLK, 16), lambda i: (i, 0)),
            full((eb, QBLK)),
            full((QBLK, eb)),
            full((16, 64)),
            full((16, 64)),
            full((8, 64)),
            full((d_f, d_f)),
            full((8, d_f)),
            full((d_f, d_out)),
            full((8, d_out)),
        ],
        out_specs=pl.BlockSpec((QBLK, d_out), lambda i: (i, 0)),
        out_shape=jax.ShapeDtypeStruct((mp, d_out), jnp.float32),
    )(xj, pj, pq, rmat, pmat, a16, b16, wpb, watt, battp, wglob, bglobp)


def kernel(x, pos, batch, W_pos, b_pos, W_att, b_att, W_glob, b_glob):
    n, d_feat = x.shape
    m = n // 4
    d_r = W_pos.shape[1]
    d_f = d_feat + d_r
    d_out = W_glob.shape[1]

    np_pad = _ceil_to(n, PSLICE)
    mp = _ceil_to(m, QCHUNK)
    e_pad = mp * K

    idx = jax.random.randint(jax.random.key(42), (m,), 0, n)
    idx_p = jnp.concatenate([idx, jnp.zeros((mp - m,), jnp.int32)])

    pos16 = jnp.concatenate([pos, jnp.zeros((n, 13), jnp.float32)], axis=1)
    pad_rows = jnp.concatenate(
        [
            jnp.full((np_pad - n, 1), 1e18, jnp.float32),
            jnp.zeros((np_pad - n, 15), jnp.float32),
        ],
        axis=1,
    )
    pos16 = jnp.concatenate([pos16, pad_rows], axis=0)

    posq16 = _sc_row_gather(np_pad, 16, mp, jnp.float32)(pos16, idx_p)

    pnr = jnp.sum(pos16 * pos16, axis=1).reshape(np_pad // PSLICE, PSLICE)
    posb = pos16.astype(jnp.bfloat16)
    nbr = _knn_call(posq16, posb, pnr, mp, np_pad)
    nbr_flat = jnp.minimum(nbr.reshape(e_pad), n - 1)

    xj, pj = _sc_edge_gather(np_pad, d_feat, 16, e_pad)(
        jnp.concatenate([x, jnp.zeros((np_pad - n, d_feat), x.dtype)]),
        pos16, nbr_flat)

    rep = np.zeros((QBLK * K, QBLK), np.float32)
    rep[np.arange(QBLK * K), np.arange(QBLK * K) // K] = 1.0
    rmat = jnp.asarray(rep)
    pmat = jnp.asarray(rep.T / K)

    a16 = jnp.zeros((16, d_r), jnp.float32).at[:3].set(W_pos[0:3] + W_pos[6:9])
    b16 = jnp.zeros((16, d_r), jnp.float32).at[:3].set(W_pos[3:6] - W_pos[6:9])
    wpb = jnp.zeros((8, d_r), jnp.float32).at[0].set(W_pos[9]).at[1].set(b_pos)
    battp = jnp.zeros((8, d_f), jnp.float32).at[0].set(b_att)
    bglobp = jnp.zeros((8, d_out), jnp.float32).at[0].set(b_glob)

    out = _msg_call(xj, pj, posq16, rmat, pmat, a16, b16, wpb, W_att, battp,
                    W_glob, bglobp, mp, d_feat, d_f, d_out)
    return out[:m]
